# baseline (speedup 1.0000x reference)
import functools

import jax
import jax.numpy as jnp
from jax import lax
from jax.experimental import pallas as pl
from jax.experimental.pallas import tpu as pltpu
from jax.experimental.pallas import tpu_sc as plsc

_TOPK = 8
_NE = 64
_NG = 8
_EPG = _NE // _NG
_TG = 4
_SCALE = 16.0

_N = 16384
_NW = 32
_CH = _N // _NW
_L = 16


def _scores_block(x_ref, w_ref, st_ref):
    x = x_ref[...]
    w = w_ref[...]
    logits = jax.lax.dot_general(
        x, w, (((1,), (1,)), ((), ())),
        preferred_element_type=jnp.float32,
        precision=jax.lax.Precision.DEFAULT,
    )
    lt = logits.T
    m = jnp.max(lt, axis=0, keepdims=True)
    e = jnp.exp(lt - m)
    s = jnp.sum(e, axis=0, keepdims=True)
    sc = e / s
    nt = sc.shape[1] // 128
    for tb in range(nt):
        st_ref[:, tb, :, :] = sc[:, 128 * tb:128 * (tb + 1)].reshape(
            _NG, _EPG, 128)


def _cswap_min(a, b):
    return jnp.minimum(a, b), jnp.maximum(a, b)


def _tree_argmax(pairs):
    while len(pairs) > 1:
        nxt = []
        for i in range(0, len(pairs) - 1, 2):
            (v1, i1), (v2, i2) = pairs[i], pairs[i + 1]
            c = v2 > v1
            nxt.append((jnp.where(c, v2, v1), jnp.where(c, i2, i1)))
        if len(pairs) % 2:
            nxt.append(pairs[-1])
        pairs = nxt
    return pairs[0]


def _route_sc_body(ch, st_hbm, idx_hbm, wgt_hbm, sv, iv, wv):
    wid = lax.axis_index("s") * 2 + lax.axis_index("c")
    base = wid * ch
    ntb = ch // 128
    for a in range(_NG):
        pltpu.sync_copy(st_hbm.at[a, pl.ds(base // 128, ntb)], sv.at[a])

    def step(t0, carry):
        col = pl.ds(t0 * _L, _L)
        tb = t0 // (128 // _L)
        c0 = (t0 % (128 // _L)) * _L
        ccol = pl.ds(c0, _L)
        cvec = c0 + lax.iota(jnp.int32, _L)
        tbv = jnp.full((_L,), 0, jnp.int32) + tb
        s = [sv[e // _EPG, tb, e % _EPG, ccol] for e in range(_NE)]

        gv = []
        for g in range(_NG):
            m1 = [jnp.maximum(s[_EPG * g + j], s[_EPG * g + j + 1])
                  for j in range(0, _EPG, 2)]
            m2 = [jnp.maximum(m1[j], m1[j + 1]) for j in range(0, 4, 2)]
            gv.append(jnp.maximum(m2[0], m2[1]))

        gidx = [jnp.full((_L,), g, jnp.int32) for g in range(_NG)]
        chosen = []
        for _ in range(_TG):
            _, bi = _tree_argmax(list(zip(gv, gidx)))
            chosen.append(bi)
            for g in range(_NG):
                gv[g] = jnp.where(bi == g, -1.0, gv[g])

        a, b = _cswap_min(chosen[0], chosen[1])
        c, d = _cswap_min(chosen[2], chosen[3])
        a, c = _cswap_min(a, c)
        b, d = _cswap_min(b, d)
        b, c = _cswap_min(b, c)
        groups = [a, b, c, d]
        gbases = [g * _EPG for g in groups]

        cand = []
        for k in range(_TG):
            for j in range(_EPG):
                val = plsc.load_gather(
                    sv, [groups[k], tbv, jnp.full((_L,), j, jnp.int32),
                         cvec])
                cand.append((val, gbases[k] + j))

        for r in range(_TOPK):
            bv, bi = _tree_argmax(cand)
            iv[tb, r, ccol] = bi
            wv[tb, r, ccol] = bv * _SCALE
            if r + 1 < _TOPK:
                cand = [(jnp.where(ci == bi, -1.0, cv), ci)
                        for cv, ci in cand]
        return carry

    lax.fori_loop(0, ch // _L, step, 0)
    pltpu.sync_copy(iv, idx_hbm.at[pl.ds(base // 128, ntb)])
    pltpu.sync_copy(wv, wgt_hbm.at[pl.ds(base // 128, ntb)])


def kernel(hidden_states, weight):
    b, sq, h = hidden_states.shape
    x = hidden_states.reshape(-1, h)
    n = x.shape[0]
    bt = 1024
    ch = n // _NW
    scores_t = pl.pallas_call(
        _scores_block,
        grid=(n // bt,),
        in_specs=[
            pl.BlockSpec((bt, h), lambda i: (i, 0)),
            pl.BlockSpec((_NE, h), lambda i: (0, 0)),
        ],
        out_specs=pl.BlockSpec((_NG, bt // 128, _EPG, 128),
                               lambda i: (0, i, 0, 0)),
        out_shape=jax.ShapeDtypeStruct((_NG, n // 128, _EPG, 128),
                                       jnp.float32),
    )(x, weight)

    route = pl.kernel(
        functools.partial(_route_sc_body, ch),
        out_type=[
            jax.ShapeDtypeStruct((n // 128, _TOPK, 128), jnp.int32),
            jax.ShapeDtypeStruct((n // 128, _TOPK, 128), jnp.float32),
        ],
        mesh=plsc.VectorSubcoreMesh(core_axis_name="c", subcore_axis_name="s",
                                    num_cores=2, num_subcores=16),
        compiler_params=pltpu.CompilerParams(use_tc_tiling_on_sc=False,
                                             needs_layout_passes=False,
                                             skip_device_barrier=True),
        scratch_types=[
            pltpu.VMEM((_NG, ch // 128, _EPG, 128), jnp.float32),
            pltpu.VMEM((ch // 128, _TOPK, 128), jnp.int32),
            pltpu.VMEM((ch // 128, _TOPK, 128), jnp.float32),
        ],
    )
    idx3, wgt3 = route(scores_t)
    idx = idx3.transpose(1, 0, 2).reshape(_TOPK, n).T
    wgt = wgt3.transpose(1, 0, 2).reshape(_TOPK, n).T
    return idx, wgt


# speedup vs baseline: 1.0193x; 1.0193x over previous
import functools

import jax
import jax.numpy as jnp
from jax import lax
from jax.experimental import pallas as pl
from jax.experimental.pallas import tpu as pltpu
from jax.experimental.pallas import tpu_sc as plsc

_TOPK = 8
_NE = 64
_NG = 8
_EPG = _NE // _NG
_TG = 4
_SCALE = 16.0

_N = 16384
_NW = 32
_CH = _N // _NW
_L = 16


def _scores_block(x_ref, w_ref, st_ref):
    x = x_ref[...]
    w = w_ref[...]
    logits = jax.lax.dot_general(
        x, w, (((1,), (1,)), ((), ())),
        preferred_element_type=jnp.float32,
        precision=jax.lax.Precision.DEFAULT,
    )
    lt = logits.T
    m = jnp.max(lt, axis=0, keepdims=True)
    e = jnp.exp(lt - m)
    s = jnp.sum(e, axis=0, keepdims=True)
    sc = e / s
    nt = sc.shape[1] // 128
    for tb in range(nt):
        st_ref[:, tb, :, :] = sc[:, 128 * tb:128 * (tb + 1)].reshape(
            _NG, _EPG, 128)


def _cswap_min(a, b):
    return jnp.minimum(a, b), jnp.maximum(a, b)


def _tree_argmax(pairs):
    while len(pairs) > 1:
        nxt = []
        for i in range(0, len(pairs) - 1, 2):
            (v1, i1), (v2, i2) = pairs[i], pairs[i + 1]
            c = v2 > v1
            nxt.append((jnp.where(c, v2, v1), jnp.where(c, i2, i1)))
        if len(pairs) % 2:
            nxt.append(pairs[-1])
        pairs = nxt
    return pairs[0]


def _route_sc_body(ch, st_hbm, idx_hbm, wgt_hbm, sv, iv, wv):
    wid = lax.axis_index("s") * 2 + lax.axis_index("c")
    base = wid * ch
    ntb = ch // 128
    for a in range(_NG):
        pltpu.sync_copy(st_hbm.at[a, pl.ds(base // 128, ntb)], sv.at[a])

    def step(t0, carry):
        tb = t0 // (128 // _L)
        c0 = (t0 % (128 // _L)) * _L
        ccol = pl.ds(c0, _L)
        cvec = c0 + lax.iota(jnp.int32, _L)
        tbv = jnp.full((_L,), 0, jnp.int32) + tb
        s = [sv[e // _EPG, tb, e % _EPG, ccol] for e in range(_NE)]

        gv = []
        for g in range(_NG):
            m1 = [jnp.maximum(s[_EPG * g + j], s[_EPG * g + j + 1])
                  for j in range(0, _EPG, 2)]
            m2 = [jnp.maximum(m1[j], m1[j + 1]) for j in range(0, 4, 2)]
            gv.append(jnp.maximum(m2[0], m2[1]))

        gidx = [jnp.full((_L,), g, jnp.int32) for g in range(_NG)]
        chosen = []
        for _ in range(_TG):
            _, bi = _tree_argmax(list(zip(gv, gidx)))
            chosen.append(bi)
            for g in range(_NG):
                gv[g] = jnp.where(bi == g, -1.0, gv[g])

        a, b = _cswap_min(chosen[0], chosen[1])
        c, d = _cswap_min(chosen[2], chosen[3])
        a, c = _cswap_min(a, c)
        b, d = _cswap_min(b, d)
        b, c = _cswap_min(b, c)
        groups = [a, b, c, d]
        gbases = [g * _EPG for g in groups]

        cand = []
        for k in range(_TG):
            for j in range(_EPG):
                val = plsc.load_gather(
                    sv, [groups[k], tbv, jnp.full((_L,), j, jnp.int32),
                         cvec])
                cand.append((val, gbases[k] + j))

        for r in range(_TOPK):
            bv, bi = _tree_argmax(cand)
            iv[tb, r, ccol] = bi
            wv[tb, r, ccol] = bv * _SCALE
            if r + 1 < _TOPK:
                cand = [(jnp.where(ci == bi, -1.0, cv), ci)
                        for cv, ci in cand]
        return carry

    lax.fori_loop(0, ch // _L, step, 0)
    pltpu.sync_copy(iv, idx_hbm.at[pl.ds(base // 128, ntb)])
    pltpu.sync_copy(wv, wgt_hbm.at[pl.ds(base // 128, ntb)])


def kernel(hidden_states, weight):
    b, sq, h = hidden_states.shape
    x = hidden_states.reshape(-1, h)
    n = x.shape[0]
    bt = 512
    ch = n // _NW
    scores_t = pl.pallas_call(
        _scores_block,
        grid=(n // bt,),
        in_specs=[
            pl.BlockSpec((bt, h), lambda i: (i, 0)),
            pl.BlockSpec((_NE, h), lambda i: (0, 0)),
        ],
        out_specs=pl.BlockSpec((_NG, bt // 128, _EPG, 128),
                               lambda i: (0, i, 0, 0)),
        out_shape=jax.ShapeDtypeStruct((_NG, n // 128, _EPG, 128),
                                       jnp.float32),
    )(x, weight)

    route = pl.kernel(
        functools.partial(_route_sc_body, ch),
        out_type=[
            jax.ShapeDtypeStruct((n // 128, _TOPK, 128), jnp.int32),
            jax.ShapeDtypeStruct((n // 128, _TOPK, 128), jnp.float32),
        ],
        mesh=plsc.VectorSubcoreMesh(core_axis_name="c", subcore_axis_name="s",
                                    num_cores=2, num_subcores=16),
        compiler_params=pltpu.CompilerParams(use_tc_tiling_on_sc=False,
                                             needs_layout_passes=False,
                                             skip_device_barrier=True),
        scratch_types=[
            pltpu.VMEM((_NG, ch // 128, _EPG, 128), jnp.float32),
            pltpu.VMEM((ch // 128, _TOPK, 128), jnp.int32),
            pltpu.VMEM((ch // 128, _TOPK, 128), jnp.float32),
        ],
    )
    idx3, wgt3 = route(scores_t)
    idx = idx3.transpose(1, 0, 2).reshape(_TOPK, n).T
    wgt = wgt3.transpose(1, 0, 2).reshape(_TOPK, n).T
    return idx, wgt
